# baseline (speedup 1.0000x reference)
import jax
import jax.numpy as jnp
from jax.experimental import pallas as pl
from jax.experimental.pallas import tpu as pltpu

_NCH = 16
_B = 16384
_CH = _B // _NCH


def _mlp_kernel(x_hbm, w1_ref, b1_ref, w2_ref, b2_ref, o_ref, xbuf, sem):
    for c in range(_NCH):
        pltpu.make_async_copy(
            x_hbm.at[pl.ds(c * _CH, _CH), :], xbuf.at[c], sem.at[c]
        ).start(priority=c % 2)
    w1 = w1_ref[...]
    b1v = b1_ref[...]
    w2 = w2_ref[...]
    b2v = b2_ref[...]
    rows_per_chunk = _CH // 128
    for c in range(_NCH):
        pltpu.make_async_copy(
            x_hbm.at[pl.ds(c * _CH, _CH), :], xbuf.at[c], sem.at[c]
        ).wait()
        h = jnp.dot(xbuf[c], w1, preferred_element_type=jnp.float32)
        h = jnp.maximum(h + b1v, 0.0)
        logit = jnp.sum(h * w2, axis=1, keepdims=True) + b2v
        dense = jnp.reshape(logit, (rows_per_chunk, 128))
        o_ref[pl.ds(c * rows_per_chunk, rows_per_chunk), :] = (
            0.5 * jnp.tanh(0.5 * dense) + 0.5
        )


@jax.jit
def kernel(x, W1, b1, W2, b2):
    B, D = x.shape
    H = W1.shape[1]
    b1r = b1.reshape(1, H)
    w2r = W2.reshape(1, H)
    b2r = b2.reshape(1, 1)
    out = pl.pallas_call(
        _mlp_kernel,
        in_specs=[
            pl.BlockSpec(memory_space=pl.ANY),
            pl.BlockSpec((D, H), lambda: (0, 0)),
            pl.BlockSpec((1, H), lambda: (0, 0)),
            pl.BlockSpec((1, H), lambda: (0, 0)),
            pl.BlockSpec((1, 1), lambda: (0, 0)),
        ],
        out_specs=pl.BlockSpec((B // 128, 128), lambda: (0, 0)),
        out_shape=jax.ShapeDtypeStruct((B // 128, 128), jnp.float32),
        scratch_shapes=[
            pltpu.VMEM((_NCH, _CH, 128), jnp.float32),
            pltpu.SemaphoreType.DMA((_NCH,)),
        ],
    )(x, W1, b1r, w2r, b2r)
    return out.reshape(B, 1)


# speedup vs baseline: 1.5859x; 1.5859x over previous
import jax
import jax.numpy as jnp
from jax.experimental import pallas as pl
from jax.experimental.pallas import tpu as pltpu

_B = 16384
_NCORE = 1
_CH = 1024
_NCH = _B // _NCORE // _CH


def _mlp_kernel(x_hbm, w1_ref, b1_ref, w2_ref, b2_ref, o_ref, xbuf, sem):
    i = pl.program_id(0)
    base = i * (_B // _NCORE)
    for c in range(_NCH):
        pltpu.make_async_copy(
            x_hbm.at[pl.ds(base + c * _CH, _CH), :], xbuf.at[c], sem.at[c]
        ).start(priority=c % 2)
    w1 = w1_ref[...]
    b1v = b1_ref[...]
    w2 = w2_ref[...]
    b2v = b2_ref[...]
    for c in range(_NCH):
        pltpu.make_async_copy(
            x_hbm.at[pl.ds(base + c * _CH, _CH), :], xbuf.at[c], sem.at[c]
        ).wait()
        h = jnp.dot(xbuf[c], w1, preferred_element_type=jnp.float32)
        h = jnp.maximum(h + b1v, 0.0)
        logit = jax.lax.dot_general(
            w2, h, (((1,), (1,)), ((), ())), preferred_element_type=jnp.float32
        )
        o_ref[pl.ds(c, 1), :] = 0.5 * jnp.tanh(0.5 * (logit + b2v)) + 0.5


@jax.jit
def kernel(x, W1, b1, W2, b2):
    B, D = x.shape
    H = W1.shape[1]
    b1r = b1.reshape(1, H)
    w2r = W2.reshape(1, H)
    b2r = b2.reshape(1, 1)
    out = pl.pallas_call(
        _mlp_kernel,
        grid=(_NCORE,),
        in_specs=[
            pl.BlockSpec(memory_space=pl.ANY),
            pl.BlockSpec((D, H), lambda i: (0, 0)),
            pl.BlockSpec((1, H), lambda i: (0, 0)),
            pl.BlockSpec((1, H), lambda i: (0, 0)),
            pl.BlockSpec((1, 1), lambda i: (0, 0)),
        ],
        out_specs=pl.BlockSpec((_NCH, _CH), lambda i: (i, 0)),
        out_shape=jax.ShapeDtypeStruct((_NCORE * _NCH, _CH), jnp.float32),
        scratch_shapes=[
            pltpu.VMEM((_NCH, _CH, 128), jnp.float32),
            pltpu.SemaphoreType.DMA((_NCH,)),
        ],
    )(x, W1, b1r, w2r, b2r)
    return out.reshape(B, 1)


# speedup vs baseline: 1.8875x; 1.1902x over previous
import jax
import jax.numpy as jnp
from jax.experimental import pallas as pl
from jax.experimental.pallas import tpu as pltpu

_B = 16384
_K = 4
_G = 8
_CH = _B // (_K * _G)


def _mlp_kernel(x0, x1, x2, x3, w1_ref, b1_ref, w2_ref, b2_ref, o_ref):
    w1 = w1_ref[...]
    b1v = b1_ref[...]
    w2 = w2_ref[...]
    b2v = b2_ref[...]
    for k, xk in enumerate((x0, x1, x2, x3)):
        h = jnp.dot(xk[...], w1, preferred_element_type=jnp.float32)
        h = jnp.maximum(h + b1v, 0.0)
        logit = jax.lax.dot_general(
            w2, h, (((1,), (1,)), ((), ())), preferred_element_type=jnp.float32
        )
        o_ref[0, pl.ds(k, 1), :] = 0.5 * jnp.tanh(0.5 * (logit + b2v)) + 0.5


@jax.jit
def kernel(x, W1, b1, W2, b2):
    B, D = x.shape
    H = W1.shape[1]
    b1r = b1.reshape(1, H)
    w2r = W2.reshape(1, H)
    b2r = b2.reshape(1, 1)
    out = pl.pallas_call(
        _mlp_kernel,
        grid=(_G,),
        in_specs=[
            pl.BlockSpec((_CH, D), lambda i, k=k: (_K * i + k, 0))
            for k in range(_K)
        ]
        + [
            pl.BlockSpec((D, H), lambda i: (0, 0)),
            pl.BlockSpec((1, H), lambda i: (0, 0)),
            pl.BlockSpec((1, H), lambda i: (0, 0)),
            pl.BlockSpec((1, 1), lambda i: (0, 0)),
        ],
        out_specs=pl.BlockSpec((1, _K, _CH), lambda i: (i, 0, 0)),
        out_shape=jax.ShapeDtypeStruct((_G, _K, _CH), jnp.float32),
    )(x, x, x, x, W1, b1r, w2r, b2r)
    return out.reshape(B, 1)
